# Initial kernel scaffold; baseline (speedup 1.0000x reference)
import jax
import jax.numpy as jnp
from jax.experimental import pallas as pl


def kernel(cls_head_0, cls_head_1, cls_head_2, cls_head_3, cls_head_4, reg_head_0, reg_head_1, reg_head_2, reg_head_3, reg_head_4, cnt_head_0, cnt_head_1, cnt_head_2, cnt_head_3, cnt_head_4, bbox_gt, cls_gt):
    raise NotImplementedError("write your pallas kernel here")


# speedup vs baseline: 2.4693x; 2.4693x over previous
import functools

import jax
import jax.numpy as jnp
import numpy as np
from jax import lax
from jax.experimental import pallas as pl
from jax.experimental.pallas import tpu as pltpu
from jax.experimental.pallas import tpu_sc as plsc

STRIDES = (8, 16, 32, 64, 128)
WINDOW = ((-1.0, 64.0), (64.0, 128.0), (128.0, 256.0), (256.0, 512.0),
          (512.0, 99999999.0))
SAMPLE_RATIO = 1.5
HW = ((64, 64), (32, 32), (16, 16), (8, 8), (4, 4))
B = 8
M = 50
BIG = 99999999.0

NW = 32
LANES = 16
PW = 1376
CW = PW // LANES
P_PAD = NW * PW
N_CHUNKS = NW * CW


def _build_constants():
    xs, ys, r2, wlo, whi, bbase = [], [], [], [], [], []
    for li, (h, w) in enumerate(HW):
        s = STRIDES[li]
        xg = (np.arange(w, dtype=np.float32) * s + s // 2)
        yg = (np.arange(h, dtype=np.float32) * s + s // 2)
        xx = np.tile(xg[None, :], (h, 1)).reshape(-1)
        yy = np.tile(yg[:, None], (1, w)).reshape(-1)
        for b in range(B):
            xs.append(xx)
            ys.append(yy)
            r2.append(np.full(h * w, 2.0 * s * SAMPLE_RATIO, np.float32))
            wlo.append(np.full(h * w, WINDOW[li][0], np.float32))
            whi.append(np.full(h * w, WINDOW[li][1], np.float32))
            bbase.append(np.full(h * w // LANES, b * M, np.int32))
    xs = np.concatenate(xs)
    npix = xs.size
    pad = P_PAD - npix

    def padded(a, v):
        return np.concatenate([a, np.full(pad, v, a.dtype)])

    xs = padded(xs, 0.0)
    ys = padded(np.concatenate(ys), 0.0)
    r2 = padded(np.concatenate(r2), -1.0)
    wlo = padded(np.concatenate(wlo), 0.0)
    whi = padded(np.concatenate(whi), -1.0)
    bb = np.concatenate(bbase)
    bb = np.concatenate([bb, np.zeros(N_CHUNKS - bb.size, np.int32)])
    return xs, ys, r2, wlo, whi, bb


_XF, _YF, _R2F, _WLOF, _WHIF, _BBASE = _build_constants()


def _sqrt16(x):
    xi = lax.bitcast_convert_type(x, jnp.int32)
    yi = jnp.int32(0x5F3759DF) - (xi >> 1)
    y = lax.bitcast_convert_type(yi, jnp.float32)
    hx = x * 0.5
    for _ in range(4):
        y = y * (1.5 - hx * y * y)
    return x * y


def _sc_body(x0_h, y0_h, x1_h, y1_h, cls_h, xf_h, yf_h, r2_h, wlo_h, whi_h,
             bb_h, lo_h, to_h, ro_h, bo_h, co_h, ko_h,
             x0v, y0v, x1v, y1v, clsv, xv, yv, r2v, wlov, whiv, bbv,
             lsv, tsv, rsv, bsv, csv, ksv):
    wid = lax.axis_index("s") * 2 + lax.axis_index("c")
    poff = wid * PW
    pltpu.sync_copy(x0_h, x0v)
    pltpu.sync_copy(y0_h, y0v)
    pltpu.sync_copy(x1_h, x1v)
    pltpu.sync_copy(y1_h, y1v)
    pltpu.sync_copy(cls_h, clsv)
    pltpu.sync_copy(bb_h, bbv)
    pltpu.sync_copy(xf_h.at[pl.ds(poff, PW)], xv)
    pltpu.sync_copy(yf_h.at[pl.ds(poff, PW)], yv)
    pltpu.sync_copy(r2_h.at[pl.ds(poff, PW)], r2v)
    pltpu.sync_copy(wlo_h.at[pl.ds(poff, PW)], wlov)
    pltpu.sync_copy(whi_h.at[pl.ds(poff, PW)], whiv)

    def chunk(i, _):
        s = i * LANES
        c = wid * CW + i
        cvec = jnp.full((LANES,), c, jnp.int32)
        bb = plsc.load_gather(bbv, (cvec,))
        px = xv[pl.ds(s, LANES)]
        py = yv[pl.ds(s, LANES)]
        r2 = r2v[pl.ds(s, LANES)]
        wl = wlov[pl.ds(s, LANES)]
        wh = whiv[pl.ds(s, LANES)]
        x2 = px + px
        y2 = py + py
        best_a = jnp.full((LANES,), BIG, jnp.float32)
        best_m = jnp.zeros((LANES,), jnp.int32)
        anyp = jnp.zeros((LANES,), jnp.bool_)
        for m in range(M):
            mi = bb + m
            bx0 = plsc.load_gather(x0v, (mi,))
            by0 = plsc.load_gather(y0v, (mi,))
            bx1 = plsc.load_gather(x1v, (mi,))
            by1 = plsc.load_gather(y1v, (mi,))
            l = px - bx0
            t = py - by0
            r = bx1 - px
            b2 = by1 - py
            dmin = jnp.minimum(jnp.minimum(l, t), jnp.minimum(r, b2))
            dmax = jnp.maximum(jnp.maximum(l, t), jnp.maximum(r, b2))
            dc = jnp.maximum(jnp.abs(x2 - (bx0 + bx1)),
                             jnp.abs(y2 - (by0 + by1)))
            pos = ((dmin > 0.0) & (dmax <= wh) & (dmax >= wl) & (dc < r2))
            area = (l + r) * (t + b2)
            am = jnp.where(pos, area, BIG)
            better = am < best_a
            best_a = jnp.where(better, am, best_a)
            best_m = jnp.where(better, jnp.int32(m), best_m)
            anyp = anyp | pos
        gi = bb + best_m
        gx0 = plsc.load_gather(x0v, (gi,))
        gy0 = plsc.load_gather(y0v, (gi,))
        gx1 = plsc.load_gather(x1v, (gi,))
        gy1 = plsc.load_gather(y1v, (gi,))
        gcls = plsc.load_gather(clsv, (gi,))
        l = px - gx0
        t = py - gy0
        r = gx1 - px
        b2 = gy1 - py
        lrmin = jnp.minimum(l, r)
        lrmax = jnp.maximum(l, r)
        tbmin = jnp.minimum(t, b2)
        tbmax = jnp.maximum(t, b2)
        ratio = jnp.maximum(lrmin * tbmin / (lrmax * tbmax + 1e-10), 0.0)
        cnt = _sqrt16(ratio + 1e-12)
        neg1 = jnp.full((LANES,), -1.0, jnp.float32)
        lsv[pl.ds(s, LANES)] = jnp.where(anyp, l, neg1)
        tsv[pl.ds(s, LANES)] = jnp.where(anyp, t, neg1)
        rsv[pl.ds(s, LANES)] = jnp.where(anyp, r, neg1)
        bsv[pl.ds(s, LANES)] = jnp.where(anyp, b2, neg1)
        csv[pl.ds(s, LANES)] = jnp.where(anyp, cnt, neg1)
        ksv[pl.ds(s, LANES)] = jnp.where(anyp, gcls, jnp.zeros((LANES,), jnp.int32))
        return 0

    lax.fori_loop(0, CW, chunk, 0)
    pltpu.sync_copy(lsv, lo_h.at[pl.ds(poff, PW)])
    pltpu.sync_copy(tsv, to_h.at[pl.ds(poff, PW)])
    pltpu.sync_copy(rsv, ro_h.at[pl.ds(poff, PW)])
    pltpu.sync_copy(bsv, bo_h.at[pl.ds(poff, PW)])
    pltpu.sync_copy(csv, co_h.at[pl.ds(poff, PW)])
    pltpu.sync_copy(ksv, ko_h.at[pl.ds(poff, PW)])


@jax.jit
def _targets(bbox_gt, cls_gt):
    x0 = bbox_gt[:, :, 0].reshape(-1)
    y0 = bbox_gt[:, :, 1].reshape(-1)
    x1 = bbox_gt[:, :, 2].reshape(-1)
    y1 = bbox_gt[:, :, 3].reshape(-1)
    cls = cls_gt.astype(jnp.int32).reshape(-1)

    mesh = plsc.VectorSubcoreMesh(core_axis_name="c", subcore_axis_name="s")
    f32 = jnp.float32
    out_type = (
        jax.ShapeDtypeStruct((P_PAD,), f32),
        jax.ShapeDtypeStruct((P_PAD,), f32),
        jax.ShapeDtypeStruct((P_PAD,), f32),
        jax.ShapeDtypeStruct((P_PAD,), f32),
        jax.ShapeDtypeStruct((P_PAD,), f32),
        jax.ShapeDtypeStruct((P_PAD,), jnp.int32),
    )
    scratch = [
        pltpu.VMEM((B * M,), f32),
        pltpu.VMEM((B * M,), f32),
        pltpu.VMEM((B * M,), f32),
        pltpu.VMEM((B * M,), f32),
        pltpu.VMEM((B * M,), jnp.int32),
        pltpu.VMEM((PW,), f32),
        pltpu.VMEM((PW,), f32),
        pltpu.VMEM((PW,), f32),
        pltpu.VMEM((PW,), f32),
        pltpu.VMEM((PW,), f32),
        pltpu.VMEM((N_CHUNKS,), jnp.int32),
        pltpu.VMEM((PW,), f32),
        pltpu.VMEM((PW,), f32),
        pltpu.VMEM((PW,), f32),
        pltpu.VMEM((PW,), f32),
        pltpu.VMEM((PW,), f32),
        pltpu.VMEM((PW,), jnp.int32),
    ]
    run = pl.kernel(_sc_body, out_type=out_type, mesh=mesh,
                    scratch_types=scratch,
                    compiler_params=pltpu.CompilerParams(
                        needs_layout_passes=False))
    lo, to, ro, bo, co, ko = run(
        x0, y0, x1, y1, cls,
        jnp.asarray(_XF), jnp.asarray(_YF), jnp.asarray(_R2F),
        jnp.asarray(_WLOF), jnp.asarray(_WHIF), jnp.asarray(_BBASE))

    cls_target, reg_target, cnt_target = [], [], []
    off = 0
    for h, w in HW:
        n = B * h * w
        seg = lambda a: lax.slice(a, (off,), (off + n,)).reshape(B, h * w)
        reg_target.append(jnp.stack(
            [seg(lo), seg(to), seg(ro), seg(bo)], axis=-1))
        cnt_target.append(seg(co)[..., None])
        cls_target.append(seg(ko)[..., None])
        off += n
    return tuple(cls_target), tuple(reg_target), tuple(cnt_target)


def kernel(cls_head_0, cls_head_1, cls_head_2, cls_head_3, cls_head_4,
           reg_head_0, reg_head_1, reg_head_2, reg_head_3, reg_head_4,
           cnt_head_0, cnt_head_1, cnt_head_2, cnt_head_3, cnt_head_4,
           bbox_gt, cls_gt):
    del cls_head_0, cls_head_1, cls_head_2, cls_head_3, cls_head_4
    del reg_head_0, reg_head_1, reg_head_2, reg_head_3, reg_head_4
    del cnt_head_0, cnt_head_1, cnt_head_2, cnt_head_3, cnt_head_4
    return _targets(bbox_gt, cls_gt)
